# Initial kernel scaffold; baseline (speedup 1.0000x reference)
import jax
import jax.numpy as jnp
from jax.experimental import pallas as pl


def kernel(x, mem_subject, mem_object, mem_verb, mem_rot, W_role, b_role, W_spawn, b_spawn, W_subj, b_subj, W_obj, b_obj, W_verb_gate, b_verb_gate, W_verb_c, b_verb_c, W_rot, b_rot, W_out_gate, b_out_gate, ln_gamma, ln_beta, role_label, rot_ptr, subject_locked, object_locked):
    raise NotImplementedError("write your pallas kernel here")


# speedup vs baseline: 6.8736x; 6.8736x over previous
import functools

import jax
import jax.numpy as jnp
from jax import lax
from jax.experimental import pallas as pl
from jax.experimental.pallas import tpu as pltpu

ROLE_SUBJ = 0
ROLE_OBJ = 1
ROLE_VERB = 2
ROLE_CONJ = 3
LN_EPS = 1e-5


def _body(x_ref, ms_ref, mo_ref, mv_ref, rot_ref,
          wrole_ref, brole_ref, wspawn_ref, bspawn_ref,
          wsubj_ref, bsubj_ref, wobj_ref, bobj_ref,
          wvg_ref, bvg_ref, wvc_ref, bvc_ref,
          wrot_ref, brot_ref, wog_ref, bog_ref,
          gamma_ref, beta_ref,
          role_ref, ptr_ref, slock_ref, olock_ref,
          xout_ref, rl_ref, sp_ref, ns_ref, no_ref, nv_ref, rout_ref):
    xb = x_ref[...]
    role = role_ref[...]
    ptr = ptr_ref[...]
    slock = slock_ref[...] != 0
    olock = olock_ref[...] != 0

    is_subj = role == ROLE_SUBJ
    is_obj = role == ROLE_OBJ
    is_verb = role == ROLE_VERB
    is_conj = role == ROLE_CONJ
    is_other = role >= (ROLE_CONJ + 1)

    rl_ref[...] = jnp.dot(xb, wrole_ref[...],
                          preferred_element_type=jnp.float32) + brole_ref[...]
    sp_ref[...] = jnp.dot(xb, wspawn_ref[...],
                          preferred_element_type=jnp.float32) + bspawn_ref[0, 0]

    subj_vec = jnp.dot(xb, wsubj_ref[...],
                       preferred_element_type=jnp.float32) + bsubj_ref[...]
    ns_ref[...] = jnp.where(is_subj & (~slock), subj_vec, ms_ref[...])
    obj_vec = jnp.dot(xb, wobj_ref[...],
                      preferred_element_type=jnp.float32) + bobj_ref[...]
    no_ref[...] = jnp.where(is_obj & (~olock), obj_vec, mo_ref[...])

    vgate = jax.nn.sigmoid(jnp.dot(xb, wvg_ref[...],
                                   preferred_element_type=jnp.float32)
                           + bvg_ref[0, 0])
    verb_vec = jnp.dot(xb, wvc_ref[...],
                       preferred_element_type=jnp.float32) + bvc_ref[...]
    mv = mv_ref[...]
    nv_ref[...] = jnp.where(is_verb, vgate * verb_vec + (1.0 - vgate) * mv, mv)

    rot_vec = jnp.dot(xb, wrot_ref[...],
                      preferred_element_type=jnp.float32) + brot_ref[...]
    nslots = rot_ref.shape[1]
    for s in range(nslots):
        val = rot_ref[:, s, :]
        val = jnp.where(is_conj, xb if s == 0 else jnp.zeros_like(val), val)
        val = jnp.where(is_other & (ptr == s), rot_vec, val)
        rout_ref[:, s, :] = val

    gate = jax.nn.sigmoid(jnp.dot(xb, wog_ref[...],
                                  preferred_element_type=jnp.float32)
                          + bog_ref[...])
    h = xb + gate * xb
    mu = jnp.mean(h, axis=-1, keepdims=True)
    var = jnp.mean((h - mu) ** 2, axis=-1, keepdims=True)
    xout_ref[...] = ((h - mu) * lax.rsqrt(var + LN_EPS) * gamma_ref[...]
                     + beta_ref[...])


def kernel(x, mem_subject, mem_object, mem_verb, mem_rot, W_role, b_role,
           W_spawn, b_spawn, W_subj, b_subj, W_obj, b_obj, W_verb_gate,
           b_verb_gate, W_verb_c, b_verb_c, W_rot, b_rot, W_out_gate,
           b_out_gate, ln_gamma, ln_beta, role_label, rot_ptr,
           subject_locked, object_locked):
    B, d = x.shape
    R = mem_rot.shape[1]
    nroles = W_role.shape[1]
    BB = 128
    grid = (B // BB,)

    def row_blk(i):
        return (i, 0)

    def rot_blk(i):
        return (i, 0, 0)

    def full(i):
        return (0, 0)

    row_spec = pl.BlockSpec((BB, d), row_blk)
    mask_spec = pl.BlockSpec((BB, 1), row_blk)
    rot_spec = pl.BlockSpec((BB, R, d), rot_blk)
    w_spec = pl.BlockSpec((d, d), full)
    vcol_spec = pl.BlockSpec((d, 1), full)
    brow_spec = pl.BlockSpec((1, d), full)
    scal_spec = pl.BlockSpec((1, 1), full)

    out_shapes = (
        jax.ShapeDtypeStruct((B, d), jnp.float32),
        jax.ShapeDtypeStruct((B, nroles), jnp.float32),
        jax.ShapeDtypeStruct((B, 1), jnp.float32),
        jax.ShapeDtypeStruct((B, d), jnp.float32),
        jax.ShapeDtypeStruct((B, d), jnp.float32),
        jax.ShapeDtypeStruct((B, d), jnp.float32),
        jax.ShapeDtypeStruct((B, R, d), jnp.float32),
    )
    out_specs = (
        row_spec,
        pl.BlockSpec((BB, nroles), row_blk),
        mask_spec,
        row_spec,
        row_spec,
        row_spec,
        rot_spec,
    )
    in_specs = [
        row_spec, row_spec, row_spec, row_spec, rot_spec,
        pl.BlockSpec((d, nroles), full), pl.BlockSpec((1, nroles), full),
        vcol_spec, scal_spec,
        w_spec, brow_spec, w_spec, brow_spec,
        vcol_spec, scal_spec, w_spec, brow_spec,
        w_spec, brow_spec, w_spec, brow_spec,
        brow_spec, brow_spec,
        mask_spec, mask_spec, mask_spec, mask_spec,
    ]

    call = pl.pallas_call(
        _body,
        grid=grid,
        in_specs=in_specs,
        out_specs=out_specs,
        out_shape=out_shapes,
    )
    x_out, role_logits, spawn_logit, new_s, new_o, new_v, rot = call(
        x, mem_subject, mem_object, mem_verb, mem_rot,
        W_role, b_role.reshape(1, nroles),
        W_spawn.reshape(d, 1), b_spawn.reshape(1, 1),
        W_subj, b_subj.reshape(1, d), W_obj, b_obj.reshape(1, d),
        W_verb_gate.reshape(d, 1), b_verb_gate.reshape(1, 1),
        W_verb_c, b_verb_c.reshape(1, d),
        W_rot, b_rot.reshape(1, d),
        W_out_gate, b_out_gate.reshape(1, d),
        ln_gamma.reshape(1, d), ln_beta.reshape(1, d),
        role_label.reshape(B, 1), rot_ptr.reshape(B, 1),
        subject_locked.astype(jnp.int32).reshape(B, 1),
        object_locked.astype(jnp.int32).reshape(B, 1),
    )
    return (x_out, role_logits, spawn_logit.reshape(B), new_s, new_o,
            new_v, rot)
